# baseline (speedup 1.0000x reference)
import functools

import jax
import jax.numpy as jnp
from jax import lax
from jax.experimental import pallas as pl
from jax.experimental.pallas import tpu as pltpu
from jax.experimental.pallas import tpu_sc as plsc

_B, _S, _K = 4096, 200, 10
_N = _S * 2 * _K
_ROWS = _B * _S
_NPOS = _ROWS * _K
_TOT = _ROWS * 2 * _K
_VOCAB = 1_000_000
_NPHASE = 2
_PROWS = _ROWS // _NPHASE
_PTOT = _TOT // _NPHASE


def _neg_indices():
    return jax.random.uniform(
        jax.random.key(12345), (_NPOS,), minval=1.0, maxval=float(_VOCAB)
    ).astype(jnp.int32)


_NC, _NS, _L = 2, 16, 16
_NW = _NC * _NS
_RPW = _PROWS // _NW
_CH = 512
_NCHUNK = _RPW // _CH
_CHP = _CH * _K
_CHO = _CH * 2 * _K
_G = _CHO // 80

_sc_mesh = plsc.VectorSubcoreMesh(core_axis_name="c", subcore_axis_name="s")

_SC_SCRATCH = [
    pltpu.VMEM_SHARED((_VOCAB,), jnp.float32),
    pltpu.VMEM((_CHO,), jnp.int32),
    pltpu.VMEM((_CHO,), jnp.int32),
    pltpu.VMEM((_CHO,), jnp.float32),
    pltpu.VMEM((_CHO,), jnp.float32),
    pltpu.VMEM((_CHO,), jnp.float32),
    pltpu.SemaphoreType.DMA,
    pltpu.SemaphoreType.DMA,
]


def _make_sc_gather(phase):
    def _sc_gather(table_hbm, tgt_hbm, neg_hbm, w_hbm,
                   table_sh, idx0, idx1, wb0, wb1, wout, sg0, sg1):
        cid = lax.axis_index("c")
        sid = lax.axis_index("s")
        wid = sid * _NC + cid

        @pl.when(sid == 0)
        def _stage():
            pltpu.async_copy(table_hbm, table_sh, sg0).wait()

        plsc.subcore_barrier()

        lane = lax.iota(jnp.int32, 16)
        pats = []
        for p in range(5):
            j = lane + p * 16
            r = j // (2 * _K)
            k = j % (2 * _K)
            pats.append(jnp.where(k < _K, r * _K + k, _CHP + r * _K + (k - _K)))

        def load_idx(c, idx_v):
            r0 = phase * _PROWS + wid * _RPW + c * _CH
            po = pl.multiple_of(r0 * _K, 8)
            pltpu.sync_copy(tgt_hbm.at[pl.ds(po, _CHP)],
                            idx_v.at[pl.ds(0, _CHP)])
            pltpu.sync_copy(neg_hbm.at[pl.ds(po, _CHP)],
                            idx_v.at[pl.ds(_CHP, _CHP)])

        def interleave_store(c, wb):
            def g_body(g, carry):
                base = g * 40
                for p in range(5):
                    vals = plsc.load_gather(wb, [pats[p] + base])
                    wout[pl.ds(g * 80 + p * 16, 16)] = vals
                return carry

            lax.fori_loop(0, _G, g_body, 0, unroll=False)
            oo = pl.multiple_of((wid * _RPW + c * _CH) * 2 * _K, 8)
            pltpu.sync_copy(wout, w_hbm.at[pl.ds(oo, _CHO)])

        load_idx(0, idx0)
        pltpu.async_copy(table_sh.at[idx0], wb0, sg0)

        def half(c, idx_a, wb_a, sg_a, idx_b, wb_b, sg_b):
            pltpu.make_async_copy(table_sh.at[idx_a], wb_a, sg_a).wait()

            @pl.when(c + 1 < _NCHUNK)
            def _prefetch():
                load_idx(c + 1, idx_b)
                pltpu.async_copy(table_sh.at[idx_b], wb_b, sg_b)

            interleave_store(c, wb_a)

        def pair_body(i, carry):
            half(2 * i, idx0, wb0, sg0, idx1, wb1, sg1)
            half(2 * i + 1, idx1, wb1, sg1, idx0, wb0, sg0)
            return carry

        lax.fori_loop(0, _NCHUNK // 2, pair_body, 0, unroll=False)
        half(_NCHUNK - 1, idx0, wb0, sg0, idx1, wb1, sg1)

    return pl.kernel(
        _sc_gather,
        out_type=jax.ShapeDtypeStruct((_PTOT,), jnp.float32),
        mesh=_sc_mesh,
        scratch_types=_SC_SCRATCH,
        compiler_params=pltpu.CompilerParams(needs_layout_passes=False),
        name=f"sc_gather_p{phase}",
    )


_sc_gather_p = [_make_sc_gather(p) for p in range(_NPHASE)]

_TBLK = 1024000
_TGRID = _PTOT // _TBLK


def _fused_body(x_ref, w_ref, o_ref):
    x = x_ref[...]
    o_ref[...] = w_ref[...] * (jnp.maximum(-x, 0.0) + jnp.log1p(jnp.exp(-jnp.abs(x))))


def _tc_fused(phase, x_flat, w_phase):
    return pl.pallas_call(
        _fused_body,
        out_shape=jax.ShapeDtypeStruct((_PTOT,), jnp.float32),
        grid=(_TGRID,),
        in_specs=[
            pl.BlockSpec((_TBLK,), lambda i, ph=phase: (i + ph * _TGRID,)),
            pl.BlockSpec((_TBLK,), lambda i: (i,)),
        ],
        out_specs=pl.BlockSpec((_TBLK,), lambda i: (i,)),
    )(x_flat, w_phase)


def kernel(logits, target, class_weights):
    tgt = target.reshape(-1)
    neg = _neg_indices()
    x_flat = logits.reshape(-1)
    outs = []
    ws = [_sc_gather_p[p](class_weights, tgt, neg) for p in range(_NPHASE)]
    for p in range(_NPHASE):
        outs.append(_tc_fused(p, x_flat, ws[p]))
    return jnp.concatenate(outs).reshape(_B, _N)


# speedup vs baseline: 1.0403x; 1.0403x over previous
import functools

import jax
import jax.numpy as jnp
from jax import lax
from jax.experimental import pallas as pl
from jax.experimental.pallas import tpu as pltpu
from jax.experimental.pallas import tpu_sc as plsc

_B, _S, _K = 4096, 200, 10
_N = _S * 2 * _K
_ROWS = _B * _S
_NPOS = _ROWS * _K
_TOT = _ROWS * 2 * _K
_VOCAB = 1_000_000


def _neg_indices():
    return jax.random.uniform(
        jax.random.key(12345), (_NPOS,), minval=1.0, maxval=float(_VOCAB)
    ).astype(jnp.int32)


_NC, _NS, _L = 2, 16, 16
_NW = _NC * _NS
_RPW = _ROWS // _NW
_CH = 512
_NCHUNK = _RPW // _CH
_CHP = _CH * _K
_CHO = _CH * 2 * _K
_G = _CHO // 80

_sc_mesh = plsc.VectorSubcoreMesh(core_axis_name="c", subcore_axis_name="s")


@functools.partial(
    pl.kernel,
    out_type=jax.ShapeDtypeStruct((_TOT,), jnp.float32),
    mesh=_sc_mesh,
    scratch_types=[
        pltpu.VMEM_SHARED((_VOCAB,), jnp.float32),
        pltpu.VMEM((_CHO,), jnp.int32),
        pltpu.VMEM((_CHO,), jnp.int32),
        pltpu.VMEM((_CHO,), jnp.float32),
        pltpu.VMEM((_CHO,), jnp.float32),
        pltpu.VMEM((_CHO,), jnp.float32),
        pltpu.SemaphoreType.DMA,
        pltpu.SemaphoreType.DMA,
    ],
    compiler_params=pltpu.CompilerParams(needs_layout_passes=False),
)
def _sc_gather(table_hbm, tgt_hbm, neg_hbm, w_hbm,
               table_sh, idx0, idx1, wb0, wb1, wout, sg0, sg1):
    cid = lax.axis_index("c")
    sid = lax.axis_index("s")
    wid = sid * _NC + cid

    @pl.when(sid == 0)
    def _stage():
        pltpu.async_copy(table_hbm, table_sh, sg0).wait()

    plsc.subcore_barrier()

    lane = lax.iota(jnp.int32, 16)
    pats = []
    for p in range(5):
        j = lane + p * 16
        r = j // (2 * _K)
        k = j % (2 * _K)
        pats.append(jnp.where(k < _K, r * _K + k, _CHP + r * _K + (k - _K)))

    def load_idx(c, idx_v):
        r0 = wid * _RPW + c * _CH
        po = pl.multiple_of(r0 * _K, 8)
        pltpu.sync_copy(tgt_hbm.at[pl.ds(po, _CHP)], idx_v.at[pl.ds(0, _CHP)])
        pltpu.sync_copy(neg_hbm.at[pl.ds(po, _CHP)], idx_v.at[pl.ds(_CHP, _CHP)])

    def interleave_store(c, wb):
        def g_body(g, carry):
            base = g * 40
            for p in range(5):
                vals = plsc.load_gather(wb, [pats[p] + base])
                wout[pl.ds(g * 80 + p * 16, 16)] = vals
            return carry

        lax.fori_loop(0, _G, g_body, 0, unroll=False)
        r0 = wid * _RPW + c * _CH
        oo = pl.multiple_of(r0 * 2 * _K, 8)
        pltpu.sync_copy(wout, w_hbm.at[pl.ds(oo, _CHO)])

    load_idx(0, idx0)
    pltpu.async_copy(table_sh.at[idx0], wb0, sg0)

    def half(c, idx_a, wb_a, sg_a, idx_b, wb_b, sg_b):
        pltpu.make_async_copy(table_sh.at[idx_a], wb_a, sg_a).wait()

        @pl.when(c + 1 < _NCHUNK)
        def _prefetch():
            load_idx(c + 1, idx_b)
            pltpu.async_copy(table_sh.at[idx_b], wb_b, sg_b)

        interleave_store(c, wb_a)

    def pair_body(i, carry):
        half(2 * i, idx0, wb0, sg0, idx1, wb1, sg1)
        half(2 * i + 1, idx1, wb1, sg1, idx0, wb0, sg0)
        return carry

    lax.fori_loop(0, _NCHUNK // 2, pair_body, 0, unroll=False)


_TBLK = 1024000


def _fused_body(x_ref, w_ref, o_ref):
    x = x_ref[...]
    o_ref[...] = w_ref[...] * (jnp.maximum(-x, 0.0) + jnp.log1p(jnp.exp(-jnp.abs(x))))


def kernel(logits, target, class_weights):
    tgt = target.reshape(-1)
    neg = _neg_indices()
    w = _sc_gather(class_weights, tgt, neg)
    out = pl.pallas_call(
        _fused_body,
        out_shape=jax.ShapeDtypeStruct((_TOT,), jnp.float32),
        grid=(_TOT // _TBLK,),
        in_specs=[
            pl.BlockSpec((_TBLK,), lambda i: (i,)),
            pl.BlockSpec((_TBLK,), lambda i: (i,)),
        ],
        out_specs=pl.BlockSpec((_TBLK,), lambda i: (i,)),
    )(logits.reshape(-1), w)
    return out.reshape(_B, _N)


# speedup vs baseline: 1.0424x; 1.0020x over previous
import functools

import jax
import jax.numpy as jnp
from jax import lax
from jax.experimental import pallas as pl
from jax.experimental.pallas import tpu as pltpu
from jax.experimental.pallas import tpu_sc as plsc

_B, _S, _K = 4096, 200, 10
_N = _S * 2 * _K
_ROWS = _B * _S
_NPOS = _ROWS * _K
_TOT = _ROWS * 2 * _K
_VOCAB = 1_000_000


def _neg_indices():
    return jax.random.uniform(
        jax.random.key(12345), (_NPOS,), minval=1.0, maxval=float(_VOCAB)
    ).astype(jnp.int32)


_NC, _NS, _L = 2, 16, 16
_NW = _NC * _NS
_RPW = _ROWS // _NW
_CH = 512
_NCHUNK = _RPW // _CH
_CHP = _CH * _K
_CHO = _CH * 2 * _K
_G = _CHO // 80

_sc_mesh = plsc.VectorSubcoreMesh(core_axis_name="c", subcore_axis_name="s")


@functools.partial(
    pl.kernel,
    out_type=jax.ShapeDtypeStruct((_TOT,), jnp.float32),
    mesh=_sc_mesh,
    scratch_types=[
        pltpu.VMEM_SHARED((_VOCAB,), jnp.float32),
        pltpu.VMEM((_CHO,), jnp.int32),
        pltpu.VMEM((_CHO,), jnp.int32),
        pltpu.VMEM((_CHO,), jnp.float32),
        pltpu.VMEM((_CHO,), jnp.float32),
        pltpu.VMEM((_CHO,), jnp.float32),
        pltpu.SemaphoreType.DMA,
        pltpu.SemaphoreType.DMA,
    ],
    compiler_params=pltpu.CompilerParams(needs_layout_passes=False),
)
def _sc_gather(table_hbm, tgt_hbm, neg_hbm, w_hbm,
               table_sh, idx0, idx1, wb0, wb1, wout, sg0, sg1):
    cid = lax.axis_index("c")
    sid = lax.axis_index("s")
    wid = sid * _NC + cid

    @pl.when(sid == 0)
    def _stage():
        pltpu.async_copy(table_hbm, table_sh, sg0).wait()

    plsc.subcore_barrier()

    lane = lax.iota(jnp.int32, 16)
    pats = []
    for p in range(5):
        j = lane + p * 16
        r = j // (2 * _K)
        k = j % (2 * _K)
        pats.append(jnp.where(k < _K, r * _K + k, _CHP + r * _K + (k - _K)))

    def load_idx(c, idx_v):
        r0 = wid * _RPW + c * _CH
        po = pl.multiple_of(r0 * _K, 8)
        pltpu.sync_copy(tgt_hbm.at[pl.ds(po, _CHP)], idx_v.at[pl.ds(0, _CHP)])
        pltpu.sync_copy(neg_hbm.at[pl.ds(po, _CHP)], idx_v.at[pl.ds(_CHP, _CHP)])

    def interleave_store(c, wb):
        def g_body(g, carry):
            base = g * 40
            for p in range(5):
                vals = plsc.load_gather(wb, [pats[p] + base])
                wout[pl.ds(g * 80 + p * 16, 16)] = vals
            return carry

        lax.fori_loop(0, _G, g_body, 0, unroll=False)
        r0 = wid * _RPW + c * _CH
        oo = pl.multiple_of(r0 * 2 * _K, 8)
        pltpu.sync_copy(wout, w_hbm.at[pl.ds(oo, _CHO)])

    load_idx(0, idx0)
    pltpu.async_copy(table_sh.at[idx0], wb0, sg0)

    def half(c, idx_a, wb_a, sg_a, idx_b, wb_b, sg_b):
        pltpu.make_async_copy(table_sh.at[idx_a], wb_a, sg_a).wait()

        @pl.when(c + 1 < _NCHUNK)
        def _prefetch():
            load_idx(c + 1, idx_b)
            pltpu.async_copy(table_sh.at[idx_b], wb_b, sg_b)

        interleave_store(c, wb_a)

    def pair_body(i, carry):
        half(2 * i, idx0, wb0, sg0, idx1, wb1, sg1)
        half(2 * i + 1, idx1, wb1, sg1, idx0, wb0, sg0)
        return carry

    lax.fori_loop(0, _NCHUNK // 2, pair_body, 0, unroll=False)
    if _NCHUNK % 2:
        half(_NCHUNK - 1, idx0, wb0, sg0, idx1, wb1, sg1)


_TBLK = 2048000


def _fused_body(x_ref, w_ref, o_ref):
    x = x_ref[...]
    o_ref[...] = w_ref[...] * (jnp.maximum(-x, 0.0) + jnp.log1p(jnp.exp(-jnp.abs(x))))


def kernel(logits, target, class_weights):
    tgt = target.reshape(-1)
    neg = _neg_indices()
    w = _sc_gather(class_weights, tgt, neg)
    out = pl.pallas_call(
        _fused_body,
        out_shape=jax.ShapeDtypeStruct((_TOT,), jnp.float32),
        grid=(_TOT // _TBLK,),
        in_specs=[
            pl.BlockSpec((_TBLK,), lambda i: (i,)),
            pl.BlockSpec((_TBLK,), lambda i: (i,)),
        ],
        out_specs=pl.BlockSpec((_TBLK,), lambda i: (i,)),
    )(logits.reshape(-1), w)
    return out.reshape(_B, _N)
